# baseline (speedup 1.0000x reference)
import functools

import jax
import jax.numpy as jnp
from jax import lax
from jax.experimental import pallas as pl
from jax.experimental.pallas import tpu as pltpu
from jax.experimental.pallas import tpu_sc as plsc

BATCH = 16384
EMB = 64
PAIR = 2 * EMB
QUAD = 4 * EMB
NROWS = 100000
NC = 2
NS = 16
NW = NC * NS
BPW = BATCH // NW
CHUNK = BPW // 4


_RB = 4000


def _repack_body(egg, emg, pg):
    pg[:, :EMB] = egg[...]
    pg[:, EMB:] = emg[...]


def _repack(egg, emg):
    in_spec = pl.BlockSpec((_RB, EMB), lambda i: (i, 0))
    out_spec = pl.BlockSpec((_RB, PAIR), lambda i: (i, 0))
    return pl.pallas_call(
        _repack_body,
        grid=(NROWS // _RB,),
        in_specs=[in_spec, in_spec],
        out_specs=out_spec,
        out_shape=jax.ShapeDtypeStruct((NROWS, PAIR), jnp.float32),
    )(egg, emg)


_mesh = plsc.VectorSubcoreMesh(core_axis_name="c", subcore_axis_name="s")

_rows_t = jax.ShapeDtypeStruct((BATCH, QUAD), jnp.float32)


def _make_side_gather():
    @functools.partial(
        pl.kernel,
        mesh=_mesh,
        out_type=_rows_t,
        scratch_types=[
            pltpu.VMEM((BPW,), jnp.int32),
            pltpu.VMEM((CHUNK, QUAD), jnp.float32),
            pltpu.VMEM((CHUNK, QUAD), jnp.float32),
            pltpu.SemaphoreType.DMA,
            pltpu.SemaphoreType.DMA,
        ],
    )
    def _side_gather(idx_hbm, e_hbm, rows_hbm,
                     idx_v, buf_a, buf_b, sem_a, sem_b):
        wid = lax.axis_index("s") * NC + lax.axis_index("c")
        base = wid * BPW
        sl = pl.ds(base, BPW)
        pltpu.sync_copy(idx_hbm.at[sl], idx_v)
        cps = [
            pltpu.async_copy(e_hbm.at[idx_v.at[pl.ds(0, CHUNK)]], buf_a, sem_a),
            pltpu.async_copy(e_hbm.at[idx_v.at[pl.ds(CHUNK, CHUNK)]], buf_b, sem_b),
        ]
        bufs = [buf_a, buf_b]
        sems = [sem_a, sem_b]
        for k in range(4):
            b = k % 2
            cps[b].wait()
            pltpu.sync_copy(bufs[b], rows_hbm.at[pl.ds(base + k * CHUNK, CHUNK)])
            if k + 2 < 4:
                cps[b] = pltpu.async_copy(
                    e_hbm.at[idx_v.at[pl.ds((k + 2) * CHUNK, CHUNK)]],
                    bufs[b], sems[b])

    return _side_gather


_gather_u = _make_side_gather()
_gather_g = _make_side_gather()


_BB = 2048


def _tc_body(ul, gl, w1u, w1g, b1r, w2, b2r, w3, b3r, wg, wm, bo, out):
    f32 = jnp.float32
    gu = ul[:, :EMB]
    mu = ul[:, EMB:PAIR]
    gg = gl[:, PAIR:3 * EMB]
    mg = gl[:, 3 * EMB:]
    h = jnp.dot(mu, w1u[...], preferred_element_type=f32)
    h = h + jnp.dot(mg, w1g[...], preferred_element_type=f32)
    h = jnp.maximum(h + b1r[...], 0.0)
    h = jnp.maximum(jnp.dot(h, w2[...], preferred_element_type=f32) + b2r[...], 0.0)
    h = jnp.maximum(jnp.dot(h, w3[...], preferred_element_type=f32) + b3r[...], 0.0)
    logit = jnp.dot(gu * gg, wg[...], preferred_element_type=f32)
    logit = logit + jnp.dot(h, wm[...], preferred_element_type=f32) + bo[...]
    out[...] = jax.nn.sigmoid(logit)


def _tc_mlp(urows, grows, w1u, w1g, b1r, w2, b2r, w3, b3r, wg, wm, bo):
    line_spec = pl.BlockSpec((_BB, QUAD), lambda i: (i, 0))

    def _full(a):
        return pl.BlockSpec(a.shape, lambda i: tuple(0 for _ in a.shape))

    return pl.pallas_call(
        _tc_body,
        grid=(BATCH // _BB,),
        in_specs=[line_spec, line_spec,
                  _full(w1u), _full(w1g), _full(b1r), _full(w2), _full(b2r),
                  _full(w3), _full(b3r), _full(wg), _full(wm), _full(bo)],
        out_specs=pl.BlockSpec((_BB, 1), lambda i: (i, 0)),
        out_shape=jax.ShapeDtypeStruct((BATCH, 1), jnp.float32),
    )(urows, grows, w1u, w1g, b1r, w2, b2r, w3, b3r, wg, wm, bo)


def kernel(user_index, game_index, E_gcf_u, E_gcf_g, E_mlp_u, E_mlp_g,
           W1, b1, W2, b2, W3, b3, Wout, bout):
    uidx = user_index.astype(jnp.int32)
    gidx = game_index.astype(jnp.int32)
    e4 = jnp.concatenate([E_gcf_u, E_mlp_u, E_gcf_g, E_mlp_g], axis=1)
    urows = _gather_u(uidx, e4)
    grows = _gather_g(gidx, e4)
    w1u = W1[:EMB]
    w1g = W1[EMB:]
    wg = Wout[:EMB]
    wm = Wout[EMB:]
    b1r = b1.reshape(1, -1)
    b2r = b2.reshape(1, -1)
    b3r = b3.reshape(1, -1)
    bo = bout.reshape(1, -1)
    return _tc_mlp(urows, grows, w1u, w1g, b1r, W2, b2r, W3, b3r, wg, wm, bo)


# speedup vs baseline: 1.0822x; 1.0822x over previous
import functools

import jax
import jax.numpy as jnp
from jax import lax
from jax.experimental import pallas as pl
from jax.experimental.pallas import tpu as pltpu
from jax.experimental.pallas import tpu_sc as plsc

BATCH = 16384
EMB = 64
PAIR = 2 * EMB
NC = 2
NS = 16
NW = NC * NS
BPW = BATCH // NW
CHUNK = BPW // 2

_mesh = plsc.VectorSubcoreMesh(core_axis_name="c", subcore_axis_name="s")

_rows_t = jax.ShapeDtypeStruct((BATCH, PAIR), jnp.float32)


@functools.partial(
    pl.kernel,
    mesh=_mesh,
    out_type=(_rows_t, _rows_t),
    scratch_types=[
        pltpu.VMEM((BPW,), jnp.int32),
        pltpu.VMEM((BPW,), jnp.int32),
        pltpu.VMEM((CHUNK, PAIR), jnp.float32),
        pltpu.VMEM((CHUNK, PAIR), jnp.float32),
        pltpu.SemaphoreType.DMA,
        pltpu.SemaphoreType.DMA,
    ],
)
def _sc_gather(uidx_hbm, gidx_hbm, eu_hbm, eg_hbm, urows_hbm, grows_hbm,
               uidx_v, gidx_v, buf_a, buf_b, sem_a, sem_b):
    wid = lax.axis_index("s") * NC + lax.axis_index("c")
    base = wid * BPW
    sl = pl.ds(base, BPW)
    pltpu.sync_copy(uidx_hbm.at[sl], uidx_v)
    pltpu.sync_copy(gidx_hbm.at[sl], gidx_v)
    cp_a = pltpu.async_copy(eu_hbm.at[uidx_v.at[pl.ds(0, CHUNK)]], buf_a, sem_a)
    cp_b = pltpu.async_copy(eu_hbm.at[uidx_v.at[pl.ds(CHUNK, CHUNK)]], buf_b, sem_b)
    cp_a.wait()
    pltpu.sync_copy(buf_a, urows_hbm.at[pl.ds(base, CHUNK)])
    cp_a = pltpu.async_copy(eg_hbm.at[gidx_v.at[pl.ds(0, CHUNK)]], buf_a, sem_a)
    cp_b.wait()
    pltpu.sync_copy(buf_b, urows_hbm.at[pl.ds(base + CHUNK, CHUNK)])
    cp_b = pltpu.async_copy(eg_hbm.at[gidx_v.at[pl.ds(CHUNK, CHUNK)]], buf_b, sem_b)
    cp_a.wait()
    pltpu.sync_copy(buf_a, grows_hbm.at[pl.ds(base, CHUNK)])
    cp_b.wait()
    pltpu.sync_copy(buf_b, grows_hbm.at[pl.ds(base + CHUNK, CHUNK)])


_BB = 2048


def _tc_body(ul, gl, w1u, w1g, b1r, w2, b2r, w3, b3r, wg, wm, bo, out):
    f32 = jnp.float32
    gu = ul[:, :EMB]
    mu = ul[:, EMB:]
    gg = gl[:, :EMB]
    mg = gl[:, EMB:]
    h = jnp.dot(mu, w1u[...], preferred_element_type=f32)
    h = h + jnp.dot(mg, w1g[...], preferred_element_type=f32)
    h = jnp.maximum(h + b1r[...], 0.0)
    h = jnp.maximum(jnp.dot(h, w2[...], preferred_element_type=f32) + b2r[...], 0.0)
    h = jnp.maximum(jnp.dot(h, w3[...], preferred_element_type=f32) + b3r[...], 0.0)
    logit = jnp.dot(gu * gg, wg[...], preferred_element_type=f32)
    logit = logit + jnp.dot(h, wm[...], preferred_element_type=f32) + bo[...]
    out[...] = jax.nn.sigmoid(logit)


def _tc_mlp(urows, grows, w1u, w1g, b1r, w2, b2r, w3, b3r, wg, wm, bo):
    line_spec = pl.BlockSpec((_BB, PAIR), lambda i: (i, 0))

    def _full(a):
        return pl.BlockSpec(a.shape, lambda i: tuple(0 for _ in a.shape))

    return pl.pallas_call(
        _tc_body,
        grid=(BATCH // _BB,),
        in_specs=[line_spec, line_spec,
                  _full(w1u), _full(w1g), _full(b1r), _full(w2), _full(b2r),
                  _full(w3), _full(b3r), _full(wg), _full(wm), _full(bo)],
        out_specs=pl.BlockSpec((_BB, 1), lambda i: (i, 0)),
        out_shape=jax.ShapeDtypeStruct((BATCH, 1), jnp.float32),
    )(urows, grows, w1u, w1g, b1r, w2, b2r, w3, b3r, wg, wm, bo)


def kernel(user_index, game_index, E_gcf_u, E_gcf_g, E_mlp_u, E_mlp_g,
           W1, b1, W2, b2, W3, b3, Wout, bout):
    uidx = user_index.astype(jnp.int32)
    gidx = game_index.astype(jnp.int32)
    eu = jnp.concatenate([E_gcf_u, E_mlp_u], axis=1)
    eg = jnp.concatenate([E_gcf_g, E_mlp_g], axis=1)
    urows, grows = _sc_gather(uidx, gidx, eu, eg)
    w1u = W1[:EMB]
    w1g = W1[EMB:]
    wg = Wout[:EMB]
    wm = Wout[EMB:]
    b1r = b1.reshape(1, -1)
    b2r = b2.reshape(1, -1)
    b3r = b3.reshape(1, -1)
    bo = bout.reshape(1, -1)
    return _tc_mlp(urows, grows, w1u, w1g, b1r, W2, b2r, W3, b3r, wg, wm, bo)


# speedup vs baseline: 1.0844x; 1.0021x over previous
import functools

import jax
import jax.numpy as jnp
from jax import lax
from jax.experimental import pallas as pl
from jax.experimental.pallas import tpu as pltpu
from jax.experimental.pallas import tpu_sc as plsc

BATCH = 16384
EMB = 64
PAIR = 2 * EMB
NC = 2
NS = 16
NW = NC * NS
BPW = BATCH // NW
CHUNK = BPW // 2

_rows_t = jax.ShapeDtypeStruct((BATCH, PAIR), jnp.float32)


@functools.cache
def _get_sc_gather():
    mesh = plsc.VectorSubcoreMesh(core_axis_name="c", subcore_axis_name="s")

    @functools.partial(
        pl.kernel,
        mesh=mesh,
        out_type=(_rows_t, _rows_t),
        scratch_types=[
            pltpu.VMEM((BPW,), jnp.int32),
            pltpu.VMEM((BPW,), jnp.int32),
            pltpu.VMEM((CHUNK, PAIR), jnp.float32),
            pltpu.VMEM((CHUNK, PAIR), jnp.float32),
            pltpu.SemaphoreType.DMA,
            pltpu.SemaphoreType.DMA,
        ],
    )
    def _sc_gather(uidx_hbm, gidx_hbm, eu_hbm, eg_hbm, urows_hbm, grows_hbm,
                   uidx_v, gidx_v, buf_a, buf_b, sem_a, sem_b):
        wid = lax.axis_index("s") * NC + lax.axis_index("c")
        base = wid * BPW
        sl = pl.ds(base, BPW)
        pltpu.sync_copy(uidx_hbm.at[sl], uidx_v)
        pltpu.sync_copy(gidx_hbm.at[sl], gidx_v)
        cp_a = pltpu.async_copy(eu_hbm.at[uidx_v.at[pl.ds(0, CHUNK)]], buf_a, sem_a)
        cp_b = pltpu.async_copy(eu_hbm.at[uidx_v.at[pl.ds(CHUNK, CHUNK)]], buf_b, sem_b)
        cp_a.wait()
        pltpu.sync_copy(buf_a, urows_hbm.at[pl.ds(base, CHUNK)])
        cp_a = pltpu.async_copy(eg_hbm.at[gidx_v.at[pl.ds(0, CHUNK)]], buf_a, sem_a)
        cp_b.wait()
        pltpu.sync_copy(buf_b, urows_hbm.at[pl.ds(base + CHUNK, CHUNK)])
        cp_b = pltpu.async_copy(eg_hbm.at[gidx_v.at[pl.ds(CHUNK, CHUNK)]], buf_b, sem_b)
        cp_a.wait()
        pltpu.sync_copy(buf_a, grows_hbm.at[pl.ds(base, CHUNK)])
        cp_b.wait()
        pltpu.sync_copy(buf_b, grows_hbm.at[pl.ds(base + CHUNK, CHUNK)])


    return _sc_gather


_BB = 2048


def _tc_body(ul, gl, w1u, w1g, b1r, w2, b2r, w3, b3r, wg, wm, bo, out):
    f32 = jnp.float32
    gu = ul[:, :EMB]
    mu = ul[:, EMB:]
    gg = gl[:, :EMB]
    mg = gl[:, EMB:]
    h = jnp.dot(mu, w1u[...], preferred_element_type=f32)
    h = h + jnp.dot(mg, w1g[...], preferred_element_type=f32)
    h = jnp.maximum(h + b1r[...], 0.0)
    h = jnp.maximum(jnp.dot(h, w2[...], preferred_element_type=f32) + b2r[...], 0.0)
    h = jnp.maximum(jnp.dot(h, w3[...], preferred_element_type=f32) + b3r[...], 0.0)
    logit = jnp.dot(gu * gg, wg[...], preferred_element_type=f32)
    logit = logit + jnp.dot(h, wm[...], preferred_element_type=f32) + bo[...]
    out[...] = jax.nn.sigmoid(logit)


def _tc_mlp(urows, grows, w1u, w1g, b1r, w2, b2r, w3, b3r, wg, wm, bo):
    line_spec = pl.BlockSpec((_BB, PAIR), lambda i: (i, 0))

    def _full(a):
        return pl.BlockSpec(a.shape, lambda i: tuple(0 for _ in a.shape))

    return pl.pallas_call(
        _tc_body,
        grid=(BATCH // _BB,),
        in_specs=[line_spec, line_spec,
                  _full(w1u), _full(w1g), _full(b1r), _full(w2), _full(b2r),
                  _full(w3), _full(b3r), _full(wg), _full(wm), _full(bo)],
        out_specs=pl.BlockSpec((_BB, 1), lambda i: (i, 0)),
        out_shape=jax.ShapeDtypeStruct((BATCH, 1), jnp.float32),
    )(urows, grows, w1u, w1g, b1r, w2, b2r, w3, b3r, wg, wm, bo)


def kernel(user_index, game_index, E_gcf_u, E_gcf_g, E_mlp_u, E_mlp_g,
           W1, b1, W2, b2, W3, b3, Wout, bout):
    uidx = user_index.astype(jnp.int32)
    gidx = game_index.astype(jnp.int32)
    eu = jnp.concatenate([E_gcf_u, E_mlp_u], axis=1)
    eg = jnp.concatenate([E_gcf_g, E_mlp_g], axis=1)
    urows, grows = _get_sc_gather()(uidx, gidx, eu, eg)
    w1u = W1[:EMB]
    w1g = W1[EMB:]
    wg = Wout[:EMB]
    wm = Wout[EMB:]
    b1r = b1.reshape(1, -1)
    b2r = b2.reshape(1, -1)
    b3r = b3.reshape(1, -1)
    bo = bout.reshape(1, -1)
    return _tc_mlp(urows, grows, w1u, w1g, b1r, W2, b2r, W3, b3r, wg, wm, bo)
